# Initial kernel scaffold; baseline (speedup 1.0000x reference)
import jax
import jax.numpy as jnp
from jax.experimental import pallas as pl


def kernel(rpn_bbox_deltas, rpn_probs, anchors):
    raise NotImplementedError("write your pallas kernel here")


# speedup vs baseline: 14.3110x; 14.3110x over previous
import functools

import jax
import jax.numpy as jnp
import numpy as np
from jax.experimental import pallas as pl
from jax.experimental.pallas import tpu as pltpu

_N = 19881
_ROWS = 160
_LANES = 128
_NP = _ROWS * _LANES
_K = 2000
_POST = 300
_OUTP = 384

_V0 = np.float32(0.1)
_V2 = np.float32(0.2)
_HALF = np.float32(0.5)
_EPS = np.float32(1e-8)
_THR = np.float32(0.7)
_BIG = np.int32(2**30)


def _nms_kernel(deltas_ref, probs_ref, anc_ref, out_ref):
    d = deltas_ref[0]
    probs = probs_ref[0]
    a0 = anc_ref[0]
    a1 = anc_ref[1]
    a2 = anc_ref[2]
    a3 = anc_ref[3]

    d0 = d[0] * _V0
    d1 = d[1] * _V0
    d2 = d[2] * _V2
    d3 = d[3] * _V2
    aw = a3 - a1
    ah = a2 - a0
    acx = a1 + _HALF * aw
    acy = a0 + _HALF * ah
    bw = jnp.exp(d3) * aw
    bh = jnp.exp(d2) * ah
    bcx = d1 * aw + acx
    bcy = d0 * ah + acy
    y1 = bcy - _HALF * bh
    x1 = bcx - _HALF * bw
    y2 = bh + y1
    x2 = bw + x1
    area = (y2 - y1) * (x2 - x1)

    pb = pltpu.bitcast(probs, jnp.int32)

    def bis_body(_, carry):
        lo, hi = carry
        mid = lo + (hi - lo) // 2
        cnt = jnp.sum((pb >= mid).astype(jnp.int32))
        ok = cnt >= _K
        return jnp.where(ok, mid, lo), jnp.where(ok, hi, mid)

    lo0 = jnp.int32(0)
    hi0 = jnp.int32(0x7F800000)
    lo, hi = jax.lax.fori_loop(0, 31, bis_body, (lo0, hi0))
    tau = lo
    m = jnp.sum((pb > tau).astype(jnp.int32))
    e = _K - m

    gt = pb > tau
    eq = pb == tau
    eqf = eq.astype(jnp.float32)
    r_iota = jax.lax.broadcasted_iota(jnp.int32, (_LANES, _LANES), 0)
    c_iota = jax.lax.broadcasted_iota(jnp.int32, (_LANES, _LANES), 1)
    tri_incl = (r_iota <= c_iota).astype(jnp.float32)
    rowcum = jnp.dot(eqf, tri_incl, preferred_element_type=jnp.float32)
    rr = jax.lax.broadcasted_iota(jnp.int32, (_ROWS, _ROWS), 0)
    cc = jax.lax.broadcasted_iota(jnp.int32, (_ROWS, _ROWS), 1)
    tri_strict = (cc < rr).astype(jnp.float32)
    rowtot = jnp.sum(eqf, axis=1, keepdims=True)
    rowpref = jnp.dot(tri_strict, rowtot, preferred_element_type=jnp.float32)
    excl = rowpref + rowcum - eqf
    alive0 = (gt | (eq & (excl < e.astype(jnp.float32)))).astype(jnp.int32)

    flat = (jax.lax.broadcasted_iota(jnp.int32, (_ROWS, _LANES), 0) * _LANES
            + jax.lax.broadcasted_iota(jnp.int32, (_ROWS, _LANES), 1))

    kiota = jax.lax.broadcasted_iota(jnp.int32, (1, _OUTP), 1)
    zrow = jnp.zeros((1, _OUTP), jnp.float32)

    def cond(carry):
        kept, alive, _, _, _, _ = carry
        return (kept < _POST) & (jnp.max(alive) > 0)

    def body(carry):
        kept, alive, oy1, ox1, oy2, ox2 = carry
        aliveb = alive > 0
        mp = jnp.max(jnp.where(aliveb, probs, np.float32(-1.0)))
        cand = aliveb & (probs == mp)
        i = jnp.min(jnp.where(cand, flat, _BIG))
        ihot = flat == i
        ihf = ihot.astype(jnp.float32)
        by1 = jnp.sum(y1 * ihf)
        bx1 = jnp.sum(x1 * ihf)
        by2 = jnp.sum(y2 * ihf)
        bx2 = jnp.sum(x2 * ihf)
        bar = jnp.sum(area * ihf)
        iy1 = jnp.maximum(by1, y1)
        ix1 = jnp.maximum(bx1, x1)
        iy2 = jnp.minimum(by2, y2)
        ix2 = jnp.minimum(bx2, x2)
        ih = jnp.maximum(iy2 - iy1, np.float32(0.0))
        iw = jnp.maximum(ix2 - ix1, np.float32(0.0))
        inter = ih * iw
        iou = inter / ((bar + area) - inter + _EPS)
        alive = jnp.where((iou > _THR) | ihot, 0, alive)
        khot = (kiota == kept).astype(jnp.float32)
        one = np.float32(1.0)
        zero = np.float32(0.0)
        oy1 = oy1 + jnp.minimum(jnp.maximum(by1, zero), one) * khot
        ox1 = ox1 + jnp.minimum(jnp.maximum(bx1, zero), one) * khot
        oy2 = oy2 + jnp.minimum(jnp.maximum(by2, zero), one) * khot
        ox2 = ox2 + jnp.minimum(jnp.maximum(bx2, zero), one) * khot
        return kept + 1, alive, oy1, ox1, oy2, ox2

    init = (jnp.int32(0), alive0, zrow, zrow, zrow, zrow)
    _, _, oy1, ox1, oy2, ox2 = jax.lax.while_loop(cond, body, init)

    out_ref[0, 0:1, :] = oy1
    out_ref[0, 1:2, :] = ox1
    out_ref[0, 2:3, :] = oy2
    out_ref[0, 3:4, :] = ox2


@jax.jit
def kernel(rpn_bbox_deltas, rpn_probs, anchors):
    b = rpn_bbox_deltas.shape[0]
    pad = _NP - _N
    deltas_t = jnp.transpose(rpn_bbox_deltas, (0, 2, 1))
    deltas_t = jnp.pad(deltas_t, ((0, 0), (0, 0), (0, pad)))
    deltas_t = deltas_t.reshape(b, 4, _ROWS, _LANES)
    probs_p = jnp.pad(rpn_probs, ((0, 0), (0, pad)), constant_values=-1.0)
    probs_p = probs_p.reshape(b, _ROWS, _LANES)
    anc_t = jnp.pad(jnp.transpose(anchors, (1, 0)), ((0, 0), (0, pad)))
    anc_t = anc_t.reshape(4, _ROWS, _LANES)

    out = pl.pallas_call(
        _nms_kernel,
        grid=(b,),
        in_specs=[
            pl.BlockSpec((1, 4, _ROWS, _LANES), lambda i: (i, 0, 0, 0)),
            pl.BlockSpec((1, _ROWS, _LANES), lambda i: (i, 0, 0)),
            pl.BlockSpec((4, _ROWS, _LANES), lambda i: (0, 0, 0)),
        ],
        out_specs=pl.BlockSpec((1, 4, _OUTP), lambda i: (i, 0, 0)),
        out_shape=jax.ShapeDtypeStruct((b, 4, _OUTP), jnp.float32),
        compiler_params=pltpu.CompilerParams(
            dimension_semantics=("arbitrary",),
        ),
    )(deltas_t, probs_p, anc_t)

    return jnp.transpose(out[:, :, :_POST], (0, 2, 1))


# speedup vs baseline: 14.3126x; 1.0001x over previous
import functools

import jax
import jax.numpy as jnp
import numpy as np
from jax.experimental import pallas as pl
from jax.experimental.pallas import tpu as pltpu

_N = 19881
_ROWS = 160
_LANES = 128
_NP = _ROWS * _LANES
_K = 2000
_POST = 300
_OUTP = 384

_V0 = np.float32(0.1)
_V2 = np.float32(0.2)
_HALF = np.float32(0.5)
_EPS = np.float32(1e-8)
_THR = np.float32(0.7)
_BIG = np.int32(2**30)


def _nms_kernel(deltas_ref, probs_ref, anc_ref, out_ref):
    d = deltas_ref[0]
    probs = probs_ref[0]
    a0 = anc_ref[0]
    a1 = anc_ref[1]
    a2 = anc_ref[2]
    a3 = anc_ref[3]

    d0 = d[0] * _V0
    d1 = d[1] * _V0
    d2 = d[2] * _V2
    d3 = d[3] * _V2
    aw = a3 - a1
    ah = a2 - a0
    acx = a1 + _HALF * aw
    acy = a0 + _HALF * ah
    bw = jnp.exp(d3) * aw
    bh = jnp.exp(d2) * ah
    bcx = d1 * aw + acx
    bcy = d0 * ah + acy
    y1 = bcy - _HALF * bh
    x1 = bcx - _HALF * bw
    y2 = bh + y1
    x2 = bw + x1
    area = (y2 - y1) * (x2 - x1)

    pb = pltpu.bitcast(probs, jnp.int32)

    def bis_body(_, carry):
        lo, hi = carry
        mid = lo + (hi - lo) // 2
        cnt = jnp.sum((pb >= mid).astype(jnp.int32))
        ok = cnt >= _K
        return jnp.where(ok, mid, lo), jnp.where(ok, hi, mid)

    lo0 = jnp.int32(0)
    hi0 = jnp.int32(0x7F800000)
    lo, hi = jax.lax.fori_loop(0, 31, bis_body, (lo0, hi0))
    tau = lo
    m = jnp.sum((pb > tau).astype(jnp.int32))
    e = _K - m

    gt = pb > tau
    eq = pb == tau
    eqf = eq.astype(jnp.float32)
    r_iota = jax.lax.broadcasted_iota(jnp.int32, (_LANES, _LANES), 0)
    c_iota = jax.lax.broadcasted_iota(jnp.int32, (_LANES, _LANES), 1)
    tri_incl = (r_iota <= c_iota).astype(jnp.float32)
    rowcum = jnp.dot(eqf, tri_incl, preferred_element_type=jnp.float32)
    rr = jax.lax.broadcasted_iota(jnp.int32, (_ROWS, _ROWS), 0)
    cc = jax.lax.broadcasted_iota(jnp.int32, (_ROWS, _ROWS), 1)
    tri_strict = (cc < rr).astype(jnp.float32)
    rowtot = jnp.sum(eqf, axis=1, keepdims=True)
    rowpref = jnp.dot(tri_strict, rowtot, preferred_element_type=jnp.float32)
    excl = rowpref + rowcum - eqf
    alive0 = (gt | (eq & (excl < e.astype(jnp.float32)))).astype(jnp.int32)

    flat = (jax.lax.broadcasted_iota(jnp.int32, (_ROWS, _LANES), 0) * _LANES
            + jax.lax.broadcasted_iota(jnp.int32, (_ROWS, _LANES), 1))

    kiota = jax.lax.broadcasted_iota(jnp.int32, (1, _OUTP), 1)
    zrow = jnp.zeros((1, _OUTP), jnp.float32)

    def cond(carry):
        kept, alive, _, _, _, _ = carry
        return (kept < _POST) & (jnp.max(alive) > 0)

    def body(carry):
        kept, alive, oy1, ox1, oy2, ox2 = carry
        aliveb = alive > 0
        mp = jnp.max(jnp.where(aliveb, probs, np.float32(-1.0)))
        cand = aliveb & (probs == mp)
        i = jnp.min(jnp.where(cand, flat, _BIG))
        ihot = flat == i
        ihf = ihot.astype(jnp.float32)
        by1 = jnp.sum(y1 * ihf)
        bx1 = jnp.sum(x1 * ihf)
        by2 = jnp.sum(y2 * ihf)
        bx2 = jnp.sum(x2 * ihf)
        bar = jnp.sum(area * ihf)
        iy1 = jnp.maximum(by1, y1)
        ix1 = jnp.maximum(bx1, x1)
        iy2 = jnp.minimum(by2, y2)
        ix2 = jnp.minimum(bx2, x2)
        ih = jnp.maximum(iy2 - iy1, np.float32(0.0))
        iw = jnp.maximum(ix2 - ix1, np.float32(0.0))
        inter = ih * iw
        iou = inter / ((bar + area) - inter + _EPS)
        alive = jnp.where((iou > _THR) | ihot, 0, alive)
        khot = (kiota == kept).astype(jnp.float32)
        one = np.float32(1.0)
        zero = np.float32(0.0)
        oy1 = oy1 + jnp.minimum(jnp.maximum(by1, zero), one) * khot
        ox1 = ox1 + jnp.minimum(jnp.maximum(bx1, zero), one) * khot
        oy2 = oy2 + jnp.minimum(jnp.maximum(by2, zero), one) * khot
        ox2 = ox2 + jnp.minimum(jnp.maximum(bx2, zero), one) * khot
        return kept + 1, alive, oy1, ox1, oy2, ox2

    init = (jnp.int32(0), alive0, zrow, zrow, zrow, zrow)
    _, _, oy1, ox1, oy2, ox2 = jax.lax.while_loop(cond, body, init)

    out_ref[0, 0:1, :] = oy1
    out_ref[0, 1:2, :] = ox1
    out_ref[0, 2:3, :] = oy2
    out_ref[0, 3:4, :] = ox2


@jax.jit
def kernel(rpn_bbox_deltas, rpn_probs, anchors):
    b = rpn_bbox_deltas.shape[0]
    pad = _NP - _N
    deltas_t = jnp.transpose(rpn_bbox_deltas, (0, 2, 1))
    deltas_t = jnp.pad(deltas_t, ((0, 0), (0, 0), (0, pad)))
    deltas_t = deltas_t.reshape(b, 4, _ROWS, _LANES)
    probs_p = jnp.pad(rpn_probs, ((0, 0), (0, pad)), constant_values=-1.0)
    probs_p = probs_p.reshape(b, _ROWS, _LANES)
    anc_t = jnp.pad(jnp.transpose(anchors, (1, 0)), ((0, 0), (0, pad)))
    anc_t = anc_t.reshape(4, _ROWS, _LANES)

    out = pl.pallas_call(
        _nms_kernel,
        grid=(b,),
        in_specs=[
            pl.BlockSpec((1, 4, _ROWS, _LANES), lambda i: (i, 0, 0, 0)),
            pl.BlockSpec((1, _ROWS, _LANES), lambda i: (i, 0, 0)),
            pl.BlockSpec((4, _ROWS, _LANES), lambda i: (0, 0, 0)),
        ],
        out_specs=pl.BlockSpec((1, 4, _OUTP), lambda i: (i, 0, 0)),
        out_shape=jax.ShapeDtypeStruct((b, 4, _OUTP), jnp.float32),
        compiler_params=pltpu.CompilerParams(
            dimension_semantics=("parallel",),
        ),
    )(deltas_t, probs_p, anc_t)

    return jnp.transpose(out[:, :, :_POST], (0, 2, 1))
